# baseline (speedup 1.0000x reference)
import functools

import jax
import jax.numpy as jnp
from jax.experimental import pallas as pl
from jax.experimental.pallas import tpu as pltpu

LANES = 128
NB = 128


def _fused_kernel(x_ref, m_ref, b_ref, sel_ref, fcw_ref, fcb_ref,
                  ts_ref, emb_ref, log_ref, *, nb):
    rows = 16 * nb
    xe = x_ref[0].reshape(rows, 96)
    xo = x_ref[1].reshape(rows, 96)
    zblk = jnp.zeros((nb, 96), jnp.bfloat16)
    xo_dn = jnp.concatenate([zblk, xo[:-nb]], axis=0)
    xe_up = jnp.concatenate([xe[nb:], zblk], axis=0)

    dot = functools.partial(jnp.dot, preferred_element_type=jnp.float32)
    m0, m1, m2 = m_ref[0], m_ref[1], m_ref[2]
    y_e = dot(xo_dn, m0) + dot(xe, m1) + dot(xo, m2)
    y_o = dot(xe, m0) + dot(xo, m1) + dot(xe_up, m2)

    z = jnp.maximum(y_e, y_o)
    zw = jnp.maximum(z[:, :512], z[:, 512:])
    pooled = jnp.maximum(zw + b_ref[...], 0.0)
    pb = pooled.astype(jnp.bfloat16)
    ts_ref[...] = jnp.swapaxes(pooled.reshape(16, nb, 512), 0, 1)
    e = dot(pb, sel_ref[...])
    emb_ref[...] = jnp.sum(e.reshape(16, nb, LANES), axis=0)

    acc = fcb_ref[...] + dot(pb[0:nb], fcw_ref[0])
    for h in range(1, 16):
        acc += dot(pb[h * nb:(h + 1) * nb], fcw_ref[h])
    log_ref[...] = acc


def kernel(conv_w, conv_b, fc_w, fc_w_pad, fc_b_pad, x):
    B = x.shape[0]
    cout = conv_w.shape[-1]
    num_classes = fc_w.shape[1]
    npad = fc_w_pad.shape[1]

    xr = (x.transpose(0, 2, 1, 3).reshape(B, 16, 2, 3 * 32)
           .transpose(2, 1, 0, 3)).astype(jnp.bfloat16)

    S = jnp.stack([jnp.eye(32, k=1 - kw, dtype=jnp.float32) for kw in range(3)])
    M = jnp.einsum('kuw,hkcd->hcuwd', S, conv_w)
    M = (M.reshape(3, 3, 32, 16, 2, cout)
          .transpose(0, 1, 2, 4, 3, 5)
          .reshape(3, 96, 32 * cout)).astype(jnp.bfloat16)
    bias_row = jnp.tile(conv_b, (16,)).reshape(1, 16 * cout)
    sel = (jnp.tile(jnp.eye(cout, LANES, dtype=jnp.float32), (16, 1))
           / 256.0).astype(jnp.bfloat16)
    fcw = fc_w_pad.reshape(16, 16 * cout, npad).astype(jnp.bfloat16)

    body = functools.partial(_fused_kernel, nb=NB)
    ts_hp, emb, logits = pl.pallas_call(
        body,
        out_shape=(jax.ShapeDtypeStruct((B, 16, 16 * cout), jnp.float32),
                   jax.ShapeDtypeStruct((B, LANES), jnp.float32),
                   jax.ShapeDtypeStruct((B, npad), jnp.float32)),
        grid=(B // NB,),
        in_specs=[
            pl.BlockSpec((2, 16, NB, 96), lambda i: (0, 0, i, 0)),
            pl.BlockSpec((3, 96, 32 * cout), lambda i: (0, 0, 0)),
            pl.BlockSpec((1, 16 * cout), lambda i: (0, 0)),
            pl.BlockSpec((16 * cout, LANES), lambda i: (0, 0)),
            pl.BlockSpec((16, 16 * cout, npad), lambda i: (0, 0, 0)),
            pl.BlockSpec((1, npad), lambda i: (0, 0)),
        ],
        out_specs=(pl.BlockSpec((NB, 16, 16 * cout), lambda i: (i, 0, 0)),
                   pl.BlockSpec((NB, LANES), lambda i: (i, 0)),
                   pl.BlockSpec((NB, npad), lambda i: (i, 0))),
        compiler_params=pltpu.CompilerParams(dimension_semantics=("parallel",)),
    )(xr, M, bias_row, sel, fcw, fc_b_pad)

    ts = ts_hp.reshape(B, 16, 16, cout)
    return emb[:, :cout], ts, logits[:, :num_classes]


# speedup vs baseline: 1.1408x; 1.1408x over previous
import functools

import jax
import jax.numpy as jnp
from jax.experimental import pallas as pl
from jax.experimental.pallas import tpu as pltpu

LANES = 128
NB = 176


def _fused_kernel(x_ref, m_ref, b_ref, sel_ref, fcw_ref, fcb_ref,
                  ts_ref, emb_ref, log_ref, *, nb):
    rows = 16 * nb
    xe = x_ref[0].reshape(rows, 96)
    xo = x_ref[1].reshape(rows, 96)
    zblk = jnp.zeros((nb, 96), jnp.bfloat16)
    xo_dn = jnp.concatenate([zblk, xo[:-nb]], axis=0)
    xe_up = jnp.concatenate([xe[nb:], zblk], axis=0)

    dot = functools.partial(jnp.dot, preferred_element_type=jnp.float32)
    m0, m1, m2 = m_ref[0], m_ref[1], m_ref[2]
    y_e = dot(xo_dn, m0) + dot(xe, m1) + dot(xo, m2)
    y_o = dot(xe, m0) + dot(xo, m1) + dot(xe_up, m2)

    z = jnp.maximum(y_e, y_o)
    zw = jnp.maximum(z[:, :512], z[:, 512:])
    pooled = jnp.maximum(zw + b_ref[...], 0.0)
    pb = pooled.astype(jnp.bfloat16)
    ts_ref[...] = pooled.reshape(16, nb, 512)
    e = dot(pb, sel_ref[...])
    emb_ref[...] = jnp.sum(e.reshape(16, nb, LANES), axis=0)

    acc = fcb_ref[...] + dot(pb[0:nb], fcw_ref[0])
    for h in range(1, 16):
        acc += dot(pb[h * nb:(h + 1) * nb], fcw_ref[h])
    log_ref[...] = acc


def kernel(conv_w, conv_b, fc_w, fc_w_pad, fc_b_pad, x):
    B = x.shape[0]
    cout = conv_w.shape[-1]
    num_classes = fc_w.shape[1]
    npad = fc_w_pad.shape[1]

    xr = (x.transpose(0, 2, 1, 3).reshape(B, 16, 2, 3 * 32)
           .transpose(2, 1, 0, 3)).astype(jnp.bfloat16)

    S = jnp.stack([jnp.eye(32, k=1 - kw, dtype=jnp.float32) for kw in range(3)])
    M = jnp.einsum('kuw,hkcd->hcuwd', S, conv_w)
    M = (M.reshape(3, 3, 32, 16, 2, cout)
          .transpose(0, 1, 2, 4, 3, 5)
          .reshape(3, 96, 32 * cout)).astype(jnp.bfloat16)
    bias_row = jnp.tile(conv_b, (16,)).reshape(1, 16 * cout)
    sel = (jnp.tile(jnp.eye(cout, LANES, dtype=jnp.float32), (16, 1))
           / 256.0).astype(jnp.bfloat16)
    fcw = fc_w_pad.reshape(16, 16 * cout, npad).astype(jnp.bfloat16)

    body = functools.partial(_fused_kernel, nb=NB)
    ts_hp, emb, logits = pl.pallas_call(
        body,
        out_shape=(jax.ShapeDtypeStruct((16, B, 16 * cout), jnp.float32),
                   jax.ShapeDtypeStruct((B, LANES), jnp.float32),
                   jax.ShapeDtypeStruct((B, npad), jnp.float32)),
        grid=(B // NB,),
        in_specs=[
            pl.BlockSpec((2, 16, NB, 96), lambda i: (0, 0, i, 0)),
            pl.BlockSpec((3, 96, 32 * cout), lambda i: (0, 0, 0)),
            pl.BlockSpec((1, 16 * cout), lambda i: (0, 0)),
            pl.BlockSpec((16 * cout, LANES), lambda i: (0, 0)),
            pl.BlockSpec((16, 16 * cout, npad), lambda i: (0, 0, 0)),
            pl.BlockSpec((1, npad), lambda i: (0, 0)),
        ],
        out_specs=(pl.BlockSpec((16, NB, 16 * cout), lambda i: (0, i, 0)),
                   pl.BlockSpec((NB, LANES), lambda i: (i, 0)),
                   pl.BlockSpec((NB, npad), lambda i: (i, 0))),
        compiler_params=pltpu.CompilerParams(dimension_semantics=("parallel",)),
    )(xr, M, bias_row, sel, fcw, fc_b_pad)

    ts = ts_hp.transpose(1, 0, 2).reshape(B, 16, 16, cout)
    return emb[:, :cout], ts, logits[:, :num_classes]
